# baseline (speedup 1.0000x reference)
import functools

import numpy as np
import jax
import jax.numpy as jnp
from jax import lax
from jax.experimental import pallas as pl
from jax.experimental.pallas import tpu as pltpu
from jax.experimental.pallas import tpu_sc as plsc

_ANCH = np.array([
    [[3.625, 2.8125], [4.875, 6.1875], [11.65625, 10.1875]],
    [[1.875, 3.8125], [3.875, 2.8125], [3.6875, 7.4375]],
    [[1.25, 1.625], [2.0, 3.75], [4.125, 2.875]],
], dtype=np.float32)
_BASE9 = np.array(
    [[10, 13], [16, 30], [33, 23], [30, 61], [62, 45], [59, 119],
     [116, 90], [156, 198], [373, 326]], dtype=np.float32)
_STRIDE = (32.0, 16.0, 8.0)
_MG = (2, 1, 0)
_IGN = 0.7
_N = 50
_NP = 64
_B = 8
_SZ = (19, 38, 76)
_NCH = 85
_R_FIN, _R_SC, _R_TEX, _R_TEY, _R_TWX, _R_TWY, _R_CLS, _R_VAL = range(8)
_R_LX, _R_LY, _R_LW, _R_LH, _R_IF, _R_JF, _R_AW, _R_AH, _R_ANY = range(8, 17)
_NROW = 24


def _tgt_stage(t5, rwh_ref, lyr, ny, nx):
    N = _N
    fnx, fny = float(nx), float(ny)
    nt = jnp.sum((jnp.sum(t5, axis=1) > 0).astype(jnp.float32))
    tio = lax.broadcasted_iota(jnp.int32, (N, 1), 0).astype(jnp.float32)
    validf = (tio < nt).astype(jnp.float32)
    lx = t5[:, 1:2] * fnx
    ly = t5[:, 2:3] * fny
    lw = t5[:, 3:4] * fnx
    lh = t5[:, 4:5] * fny
    rw = rwh_ref[0:1, :]
    rh = rwh_ref[1:2, :]
    bw = jnp.minimum(lw, rw)
    bh = jnp.minimum(lh, rh)
    en9 = ((bw > 0.0) & (bh > 0.0)).astype(jnp.float32)
    inter9 = bw * bh * en9
    iou9 = inter9 / (lw * lh + rw * rh - inter9)
    rowmax = jnp.max(iou9, axis=1, keepdims=True)
    i9 = lax.broadcasted_iota(jnp.int32, (N, 9), 1)
    best = jnp.min(jnp.where(iou9 == rowmax, i9, 9), axis=1, keepdims=True)
    m = (best // 3) == _MG[lyr]
    best3 = (best - 3 * (best // 3)).astype(jnp.float32)
    okf = ((tio < nt) & m).astype(jnp.float32)
    any_m = jnp.max(okf)
    return validf, lx, ly, lw, lh, best3, okf, any_m, tio


def _pre_body(tgt_ref, rwh_ref, dat_ref, idx_ref, *, lyr):
    N = _N
    ny = nx = _SZ[lyr]
    fnx, fny = float(nx), float(ny)
    b = pl.program_id(0)
    t5 = tgt_ref[0]
    validf, lx, ly, lw, lh, best3, okf, any_m, tio = _tgt_stage(
        t5, rwh_ref, lyr, ny, nx)

    i_f = jnp.floor(lx)
    j_f = jnp.floor(ly)
    cell = (best3 * fny + j_f) * fnx + i_f

    ir = lax.broadcasted_iota(jnp.int32, (N, N), 0)
    ic = lax.broadcasted_iota(jnp.int32, (N, N), 1)
    eyef = (ir == ic).astype(jnp.float32)
    cell_r = jnp.sum(eyef * cell, axis=0, keepdims=True)
    ok_r = jnp.sum(eyef * okf, axis=0, keepdims=True)
    later_same = ((cell == cell_r) & (ok_r > 0.0) & (ir < ic)).astype(jnp.float32)
    ow = jnp.max(later_same, axis=1, keepdims=True)
    fin = okf * (1.0 - ow)

    sc = jnp.sqrt(2.0 - lw * lh / (fnx * fny))
    tex = lx - i_f
    tey = ly - j_f
    a = _ANCH[lyr]
    is1 = best3 == 1.0
    is2 = best3 == 2.0
    aw = jnp.where(is2, a[2, 0], jnp.where(is1, a[1, 0], a[0, 0]))
    ah = jnp.where(is2, a[2, 1], jnp.where(is1, a[1, 1], a[0, 1]))
    twx = jnp.log(lw / aw + 1e-16)
    twy = jnp.log(lh / ah + 1e-16)
    clsf = jnp.floor(t5[:, 0:1])

    def row(v):
        r = jnp.sum(eyef * v, axis=0, keepdims=True)
        return jnp.concatenate([r, jnp.zeros((1, _NP - N), jnp.float32)], axis=1)

    anyrow = jnp.full((1, _NP), any_m, jnp.float32)
    zrow = jnp.zeros((1, _NP), jnp.float32)
    rows = [row(fin), row(sc), row(tex), row(tey), row(twx), row(twy),
            row(clsf), row(validf), row(lx), row(ly), row(lw), row(lh),
            row(i_f), row(j_f), row(aw), row(ah), anyrow]
    rows += [zrow] * (_NROW - len(rows))
    dat_ref[0] = jnp.concatenate(rows, axis=0)

    off = (((b.astype(jnp.float32) * 3.0 + best3) * fny + j_f) * fnx + i_f)
    off_row = row(off)
    chi = lax.broadcasted_iota(jnp.int32, (_NCH, 1), 0).astype(jnp.float32)
    idx_ref[0] = (off_row * float(_NCH) + chi).astype(jnp.int32)


def _preproc(tgt, rwh, lyr):
    return pl.pallas_call(
        functools.partial(_pre_body, lyr=lyr),
        grid=(_B,),
        in_specs=[
            pl.BlockSpec((1, _N, 5), lambda b: (b, 0, 0)),
            pl.BlockSpec((2, 9), lambda b: (0, 0)),
        ],
        out_specs=[
            pl.BlockSpec((1, _NROW, _NP), lambda b: (b, 0, 0)),
            pl.BlockSpec((1, _NCH, _NP), lambda b: (b, 0, 0)),
        ],
        out_shape=[
            jax.ShapeDtypeStruct((_B, _NROW, _NP), jnp.float32),
            jax.ShapeDtypeStruct((_B, _NCH, _NP), jnp.int32),
        ],
    )(tgt, rwh)


_NIDX = _B * _NCH * _NP
_NTILE = 32
_PERT = _NIDX // _NTILE
_CH = 80


def _sc_gather(p0f, p1f, p2f, i0, i1, i2):
    mesh = plsc.VectorSubcoreMesh(core_axis_name="c", subcore_axis_name="s")

    @functools.partial(
        pl.kernel, mesh=mesh,
        out_type=[jax.ShapeDtypeStruct((_NIDX,), jnp.float32)] * 3,
        scratch_types=[
            pltpu.VMEM((_PERT,), jnp.int32),
            pltpu.VMEM((_PERT,), jnp.float32),
            pltpu.SemaphoreType.DMA,
        ],
    )
    def k(t0, t1, t2, j0, j1, j2, o0, o1, o2, idxv, rowsv, sem):
        wid = lax.axis_index("s") * 2 + lax.axis_index("c")
        base = wid * _PERT
        for tbl, jdx, out in ((t0, j0, o0), (t1, j1, o1), (t2, j2, o2)):
            pltpu.sync_copy(jdx.at[pl.ds(base, _PERT)], idxv)
            cps = [
                pltpu.async_copy(
                    tbl.at[idxv.at[pl.ds(kk * _CH, _CH)]],
                    rowsv.at[pl.ds(kk * _CH, _CH)], sem)
                for kk in range(_PERT // _CH)
            ]
            for cp in cps:
                cp.wait()
            pltpu.sync_copy(rowsv, out.at[pl.ds(base, _PERT)])

    return k(p0f, p1f, p2f, i0, i1, i2)


def _sparse_body(g0_ref, g1_ref, g2_ref, d0_ref, d1_ref, d2_ref, out_ref):
    b = pl.program_id(0)
    clamp = lambda z: jnp.maximum(z, -100.0)
    NP = _NP
    ir = lax.broadcasted_iota(jnp.int32, (NP, NP), 0)
    ic = lax.broadcasted_iota(jnp.int32, (NP, NP), 1)
    eyef = (ir == ic).astype(jnp.float32)
    chi = lax.broadcasted_iota(jnp.int32, (80, 1), 0).astype(jnp.float32)

    def col(vrow):
        return jnp.sum(eyef * vrow, axis=1, keepdims=True)

    total = jnp.zeros((), jnp.float32)
    for g_ref, d_ref in ((g0_ref, d0_ref), (g1_ref, d1_ref), (g2_ref, d2_ref)):
        g = g_ref[0]
        d = d_ref[0]
        fin = d[_R_FIN:_R_FIN + 1, :]
        sc = d[_R_SC:_R_SC + 1, :]
        tex = d[_R_TEX:_R_TEX + 1, :]
        tey = d[_R_TEY:_R_TEY + 1, :]
        twx = d[_R_TWX:_R_TWX + 1, :]
        twy = d[_R_TWY:_R_TWY + 1, :]
        clsf = d[_R_CLS:_R_CLS + 1, :]
        anym = d[_R_ANY:_R_ANY + 1, :]
        px = g[0:1, :]
        py = g[1:2, :]
        pw = g[2:3, :]
        ph = g[3:4, :]
        pobj = g[4:5, :]
        cx = px + d[_R_IF:_R_IF + 1, :]
        cy = py + d[_R_JF:_R_JF + 1, :]
        pwv = jnp.exp(pw) * d[_R_AW:_R_AW + 1, :]
        phv = jnp.exp(ph) * d[_R_AH:_R_AH + 1, :]
        lxT = col(d[_R_LX:_R_LX + 1, :])
        lyT = col(d[_R_LY:_R_LY + 1, :])
        lwT = col(d[_R_LW:_R_LW + 1, :])
        lhT = col(d[_R_LH:_R_LH + 1, :])
        valT = col(d[_R_VAL:_R_VAL + 1, :])
        wx = (jnp.minimum(cx + 0.5 * pwv, lxT + 0.5 * lwT)
              - jnp.maximum(cx - 0.5 * pwv, lxT - 0.5 * lwT))
        wy = (jnp.minimum(cy + 0.5 * phv, lyT + 0.5 * lhT)
              - jnp.maximum(cy - 0.5 * phv, lyT - 0.5 * lhT))
        enp = ((wx > 0.0) & (wy > 0.0)).astype(jnp.float32)
        interp = wx * wy * enp
        ioup = interp / (pwv * phv + lwT * lhT - interp) * valT
        maxiou = jnp.max(ioup, axis=0, keepdims=True)
        ign = ((maxiou > _IGN) & (anym > 0.0)).astype(jnp.float32)

        lxy = (-(tex * clamp(jnp.log(px)) + (1.0 - tex) * clamp(jnp.log(1.0 - px)))
               - (tey * clamp(jnp.log(py)) + (1.0 - tey) * clamp(jnp.log(1.0 - py)))
               ) * sc * sc
        lwh = ((pw * sc - twx * sc) ** 2 + (ph * sc - twy * sc) ** 2) * 0.5
        lobj = (-clamp(jnp.log(pobj))
                + (1.0 - ign) * clamp(jnp.log(1.0 - pobj)))
        T = (chi == clsf).astype(jnp.float32)
        P = g[5:85, :]
        lcls = jnp.sum(
            -(T * clamp(jnp.log(P)) + (1.0 - T) * clamp(jnp.log(1.0 - P))),
            axis=0, keepdims=True)
        total = total + jnp.sum((lxy + lwh + lobj + lcls) * fin)

    @pl.when(b == 0)
    def _():
        out_ref[...] = jnp.zeros((1, 1), jnp.float32)

    out_ref[...] = out_ref[...] + total


def _sparse_loss(g0, g1, g2, d0, d1, d2):
    gspec = pl.BlockSpec((1, _NCH, _NP), lambda b: (b, 0, 0))
    dspec = pl.BlockSpec((1, _NROW, _NP), lambda b: (b, 0, 0))
    out = pl.pallas_call(
        _sparse_body,
        grid=(_B,),
        in_specs=[gspec, gspec, gspec, dspec, dspec, dspec],
        out_specs=pl.BlockSpec((1, 1), lambda b: (0, 0)),
        out_shape=jax.ShapeDtypeStruct((1, 1), jnp.float32),
    )(g0, g1, g2, d0, d1, d2)
    return out[0, 0]


def _dense_body(pred_ref, tgt_ref, rwh_ref, out_ref, *, lyr):
    ny = nx = _SZ[lyr]
    C = 3 * ny * nx
    b = pl.program_id(0)
    t5 = tgt_ref[0]
    validf, lx, ly, lw, lh, best3, okf, any_m, tio = _tgt_stage(
        t5, rwh_ref, lyr, ny, nx)

    pred = pred_ref[0]
    px = pred[0:1, :]
    py = pred[1:2, :]
    pobj = pred[4:5, :]
    ci = lax.broadcasted_iota(jnp.int32, (1, C), 1)
    ii = (ci % nx).astype(jnp.float32)
    jj = ((ci // nx) % ny).astype(jnp.float32)
    ai = ci // (nx * ny)
    a = _ANCH[lyr]
    awc = jnp.where(ai == 2, a[2, 0], jnp.where(ai == 1, a[1, 0], a[0, 0]))
    ahc = jnp.where(ai == 2, a[2, 1], jnp.where(ai == 1, a[1, 1], a[0, 1]))
    cx = px + ii
    cy = py + jj
    pwv = jnp.exp(pred[2:3, :]) * awc
    phv = jnp.exp(pred[3:4, :]) * ahc
    wx = (jnp.minimum(cx + 0.5 * pwv, lx + 0.5 * lw)
          - jnp.maximum(cx - 0.5 * pwv, lx - 0.5 * lw))
    wy = (jnp.minimum(cy + 0.5 * phv, ly + 0.5 * lh)
          - jnp.maximum(cy - 0.5 * phv, ly - 0.5 * lh))
    enp = ((wx > 0.0) & (wy > 0.0)).astype(jnp.float32)
    interp = wx * wy * enp
    ioup = interp / (pwv * phv + lw * lh - interp) * validf
    maxiou = jnp.max(ioup, axis=0, keepdims=True)
    notign = 1.0 - ((maxiou > _IGN) & (any_m > 0.0)).astype(jnp.float32)
    lobj = notign * (-jnp.maximum(jnp.log(1.0 - pobj), -100.0))
    partial = jnp.sum(lobj)

    @pl.when(b == 0)
    def _():
        out_ref[...] = jnp.zeros((1, 1), jnp.float32)

    out_ref[...] = out_ref[...] + partial


def _dense_obj(pred5, tgt, rwh, lyr):
    C = 3 * _SZ[lyr] * _SZ[lyr]
    out = pl.pallas_call(
        functools.partial(_dense_body, lyr=lyr),
        grid=(_B,),
        in_specs=[
            pl.BlockSpec((1, 5, C), lambda b: (b, 0, 0)),
            pl.BlockSpec((1, _N, 5), lambda b: (b, 0, 0)),
            pl.BlockSpec((2, 9), lambda b: (0, 0)),
        ],
        out_specs=pl.BlockSpec((1, 1), lambda b: (0, 0)),
        out_shape=jax.ShapeDtypeStruct((1, 1), jnp.float32),
    )(pred5, tgt, rwh)
    return out[0, 0]


def kernel(pred0, pred1, pred2, targets):
    tgt = targets.astype(jnp.float32)
    preds = (pred0, pred1, pred2)
    rwhs = [jnp.asarray((_BASE9 / _STRIDE[l]).T) for l in range(3)]
    dats, idxs = [], []
    for l in range(3):
        d, i = _preproc(tgt, rwhs[l], l)
        dats.append(d)
        idxs.append(i)
    flats = [p.reshape(-1) for p in preds]
    g0, g1, g2 = _sc_gather(flats[0], flats[1], flats[2],
                            idxs[0].reshape(-1), idxs[1].reshape(-1),
                            idxs[2].reshape(-1))
    gs = [g.reshape(_B, _NCH, _NP) for g in (g0, g1, g2)]
    sp = _sparse_loss(gs[0], gs[1], gs[2], dats[0], dats[1], dats[2])
    total = sp
    for l in range(3):
        C = 3 * _SZ[l] * _SZ[l]
        p5 = jnp.transpose(preds[l][..., :5], (0, 4, 1, 2, 3)).reshape(_B, 5, C)
        total = total + _dense_obj(p5, tgt, rwhs[l], l)
    return total
